# baseline (speedup 1.0000x reference)
import functools

import jax
import jax.numpy as jnp
from jax import lax
from jax.experimental import pallas as pl
from jax.experimental.pallas import tpu as pltpu
from jax.experimental.pallas import tpu_sc as plsc

B, S, H = 16, 4096, 512

_mesh = plsc.VectorSubcoreMesh(core_axis_name="c", subcore_axis_name="s")


@functools.partial(
    pl.kernel,
    mesh=_mesh,
    out_type=jax.ShapeDtypeStruct((B, H), jnp.float32),
    scratch_types=[
        pltpu.VMEM((B,), jnp.int32),
        pltpu.VMEM((B, H), jnp.float32),
        pltpu.SemaphoreType.DMA,
    ],
)
def _gather_last(seq_hbm, len_hbm, out_hbm, idx_v, rows_v, sem):
    cid = lax.axis_index("c")
    sid = lax.axis_index("s")

    @pl.when((cid == 0) & (sid == 0))
    def _():
        pltpu.sync_copy(len_hbm, idx_v)
        lengths = idx_v[...]
        idx_v[...] = ((lengths - 1) & (S - 1)) + lax.iota(jnp.int32, B) * S
        pltpu.async_copy(seq_hbm.at[idx_v], rows_v, sem).wait()
        pltpu.sync_copy(rows_v, out_hbm)


def kernel(sequence, lengths):
    seq_flat = sequence.reshape(B * S, H)
    return _gather_last(seq_flat, lengths.astype(jnp.int32))


# speedup vs baseline: 1.1356x; 1.1356x over previous
import functools

import jax
import jax.numpy as jnp
from jax import lax
from jax.experimental import pallas as pl
from jax.experimental.pallas import tpu as pltpu
from jax.experimental.pallas import tpu_sc as plsc

B, S, H = 16, 4096, 512

_mesh = plsc.ScalarSubcoreMesh(axis_name="c", num_cores=1)


@functools.partial(
    pl.kernel,
    mesh=_mesh,
    out_type=jax.ShapeDtypeStruct((B, H), jnp.float32),
    scratch_types=[
        pltpu.SMEM((B,), jnp.int32),
        pltpu.SemaphoreType.DMA,
    ],
)
def _gather_last(seq_hbm, len_hbm, out_hbm, len_s, sem):
    pltpu.sync_copy(len_hbm, len_s)
    copies = []
    for b in range(B):
        row = (len_s[b] - 1) & (S - 1)
        copies.append(
            pltpu.make_async_copy(seq_hbm.at[b, row], out_hbm.at[b], sem)
        )
        copies[-1].start()
    for c in copies:
        c.wait()


def kernel(sequence, lengths):
    return _gather_last(sequence, lengths.astype(jnp.int32))


# speedup vs baseline: 7.2356x; 6.3719x over previous
import jax
import jax.numpy as jnp
from jax.experimental import pallas as pl
from jax.experimental.pallas import tpu as pltpu

B, S, H = 16, 4096, 512


def _body(len_ref, seq_ref, out_ref, sem):
    copies = []
    for b in range(B):
        row = (len_ref[b] - 1) & (S - 1)
        c = pltpu.make_async_copy(seq_ref.at[b, row], out_ref.at[b], sem)
        c.start()
        copies.append(c)
    for c in copies:
        c.wait()


def kernel(sequence, lengths):
    return pl.pallas_call(
        _body,
        out_shape=jax.ShapeDtypeStruct((B, H), jnp.float32),
        in_specs=[
            pl.BlockSpec(memory_space=pltpu.MemorySpace.SMEM),
            pl.BlockSpec(memory_space=pl.ANY),
        ],
        out_specs=pl.BlockSpec(memory_space=pl.ANY),
        scratch_shapes=[pltpu.SemaphoreType.DMA],
    )(lengths.astype(jnp.int32), sequence)
